# baseline (speedup 1.0000x reference)
import functools

import jax
import jax.numpy as jnp
from jax import lax
from jax.experimental import pallas as pl
from jax.experimental.pallas import tpu as pltpu
from jax.experimental.pallas import tpu_sc as plsc

_B = 16384
_D = 16
_NC = 2
_NS = 16
_NW = _NC * _NS
_BPW = _B // _NW
_CH = 128
_NCH = _BPW // _CH


def _gmf_body(utt, itt, uids, iids, out_hbm, uidx_v, iidx_v, u_v, i_v, sem):
    wid = lax.axis_index("s") * _NC + lax.axis_index("c")
    base = wid * _BPW

    pltpu.sync_copy(uids.at[wid], uidx_v)
    pltpu.sync_copy(iids.at[wid], iidx_v)

    copies = []
    for j in range(_NCH):
        for c in range(_D):
            copies.append(pltpu.async_copy(
                utt.at[c].at[uidx_v.at[j]],
                u_v.at[c, pl.ds(j * _CH, _CH)], sem))
            copies.append(pltpu.async_copy(
                itt.at[c].at[iidx_v.at[j]],
                i_v.at[c, pl.ds(j * _CH, _CH)], sem))
    for cp in copies:
        cp.wait()

    def mul_block(k, carry):
        for c in range(_D):
            u_v[c, pl.ds(k * 16, 16)] = (
                u_v[c, pl.ds(k * 16, 16)] * i_v[c, pl.ds(k * 16, 16)])
        return carry

    lax.fori_loop(0, _BPW // 16, mul_block, 0)

    pltpu.sync_copy(u_v, out_hbm.at[:, pl.ds(base, _BPW)])


@jax.jit
def _gmf(utt, itt, uids, iids):
    run = functools.partial(
        pl.kernel,
        mesh=plsc.VectorSubcoreMesh(core_axis_name="c", subcore_axis_name="s"),
        out_type=jax.ShapeDtypeStruct((_D, _B), jnp.float32),
        scratch_types=[
            pltpu.VMEM((_NCH, _CH), jnp.int32),
            pltpu.VMEM((_NCH, _CH), jnp.int32),
            pltpu.VMEM((_D, _BPW), jnp.float32),
            pltpu.VMEM((_D, _BPW), jnp.float32),
            pltpu.SemaphoreType.DMA,
        ],
        compiler_params=pltpu.CompilerParams(use_tc_tiling_on_sc=False),
    )(_gmf_body)
    return run(utt, itt, uids, iids)


def kernel(user_ids, item_ids, user_table, item_table):
    uids = user_ids.astype(jnp.int32).reshape(_NW, _NCH, _CH)
    iids = item_ids.astype(jnp.int32).reshape(_NW, _NCH, _CH)
    out_t = _gmf(user_table.T, item_table.T, uids, iids)
    return out_t.T


# speedup vs baseline: 18.8083x; 18.8083x over previous
import functools

import jax
import jax.numpy as jnp
from jax import lax
from jax.experimental import pallas as pl
from jax.experimental.pallas import tpu as pltpu
from jax.experimental.pallas import tpu_sc as plsc

_B = 16384
_D = 16
_NC = 2
_NS = 16
_NW = _NC * _NS
_BPW = _B // _NW
_G = 8
_NG = _BPW // _G
_IDP = 640


def _fire(g, table, idvec, panels, sem):
    for lane in range(_G):
        an_id = idvec[lane]
        k0 = pl.multiple_of(an_id & -128, 128)
        pltpu.async_copy(
            table.at[:, pl.ds(k0, 128)], panels.at[lane], sem)


def _drain(table, panels, sem):
    for lane in range(_G):
        pltpu.make_async_copy(
            table.at[:, pl.ds(0, 128)], panels.at[lane], sem).wait()


def _process(gp, uvec, ivec, upanels, ipanels, stage, iota16):
    chunk = jnp.full((16,), gp >> 4, jnp.int32)
    colbase = (gp * _G) & 127
    for lane in range(_G):
        lu = jnp.full((16,), uvec[lane] & 127, jnp.int32)
        li = jnp.full((16,), ivec[lane] & 127, jnp.int32)
        ucol = plsc.load_gather(upanels.at[lane], [iota16, lu])
        icol = plsc.load_gather(ipanels.at[lane], [iota16, li])
        col = jnp.full((16,), colbase + lane, jnp.int32)
        plsc.store_scatter(stage, [chunk, iota16, col], ucol * icol)


def _gmf_body(utt, itt, uids, iids, out_hbm,
              uidx_v, iidx_v, pu, pi, stage, sem0, sem1):
    wid = lax.axis_index("s") * _NC + lax.axis_index("c")
    base = wid * _BPW

    pltpu.sync_copy(uids.at[wid], uidx_v)
    pltpu.sync_copy(iids.at[wid], iidx_v)

    iota16 = lax.iota(jnp.int32, 16)
    sems = (sem0, sem1)

    def step(t, carry):
        for half in range(2):
            g = 2 * t + half
            uvec = uidx_v[pl.ds(g * _G, 16)]
            ivec = iidx_v[pl.ds(g * _G, 16)]
            _fire(g, utt, uvec, pu.at[half], sems[half])
            _fire(g, itt, ivec, pi.at[half], sems[half])

            prev = 1 - half

            @pl.when(g > 0)
            def _():
                gp = g - 1
                _drain(utt, pu.at[prev], sems[prev])
                _drain(itt, pi.at[prev], sems[prev])
                upvec = uidx_v[pl.ds(gp * _G, 16)]
                ipvec = iidx_v[pl.ds(gp * _G, 16)]
                _process(gp, upvec, ipvec, pu.at[prev], pi.at[prev],
                         stage, iota16)
        return carry

    lax.fori_loop(0, _NG // 2, step, 0)

    _drain(utt, pu.at[1], sems[1])
    _drain(itt, pi.at[1], sems[1])
    upvec = uidx_v[pl.ds((_NG - 1) * _G, 16)]
    ipvec = iidx_v[pl.ds((_NG - 1) * _G, 16)]
    _process(_NG - 1, upvec, ipvec, pu.at[1], pi.at[1], stage, iota16)

    for ch in range(4):
        pltpu.sync_copy(
            stage.at[ch],
            out_hbm.at[:, pl.ds(pl.multiple_of(base + ch * 128, 128), 128)])


@jax.jit
def _gmf(utt, itt, uids, iids):
    run = functools.partial(
        pl.kernel,
        mesh=plsc.VectorSubcoreMesh(core_axis_name="c", subcore_axis_name="s"),
        out_type=jax.ShapeDtypeStruct((_D, _B), jnp.float32),
        scratch_types=[
            pltpu.VMEM((_IDP,), jnp.int32),
            pltpu.VMEM((_IDP,), jnp.int32),
            pltpu.VMEM((2, _G, _D, 128), jnp.float32),
            pltpu.VMEM((2, _G, _D, 128), jnp.float32),
            pltpu.VMEM((4, _D, 128), jnp.float32),
            pltpu.SemaphoreType.DMA,
            pltpu.SemaphoreType.DMA,
        ],
        compiler_params=pltpu.CompilerParams(needs_layout_passes=False),
    )(_gmf_body)
    return run(utt, itt, uids, iids)


def kernel(user_ids, item_ids, user_table, item_table):
    uids = user_ids.astype(jnp.int32).reshape(_NW, _BPW)
    iids = item_ids.astype(jnp.int32).reshape(_NW, _BPW)
    uids = jnp.pad(uids, ((0, 0), (0, _IDP - _BPW)))
    iids = jnp.pad(iids, ((0, 0), (0, _IDP - _BPW)))
    out_t = _gmf(user_table.T, item_table.T, uids, iids)
    return out_t.T


# speedup vs baseline: 18.8422x; 1.0018x over previous
import functools

import jax
import jax.numpy as jnp
from jax import lax
from jax.experimental import pallas as pl
from jax.experimental.pallas import tpu as pltpu
from jax.experimental.pallas import tpu_sc as plsc

_B = 16384
_D = 16
_NC = 2
_NS = 16
_NW = _NC * _NS
_BPW = _B // _NW
_G = 8
_NG = _BPW // _G
_IDP = 640


def _fire(g, table, idvec, panels, sem):
    for lane in range(_G):
        an_id = idvec[lane]
        k0 = pl.multiple_of(an_id & -128, 128)
        pltpu.async_copy(
            table.at[:, pl.ds(k0, 128)], panels.at[lane], sem)


def _drain(table, panels, sem):
    for lane in range(_G):
        pltpu.make_async_copy(
            table.at[:, pl.ds(0, 128)], panels.at[lane], sem).wait()


def _process(gp, uvec, ivec, upanels, ipanels, stage, iota16):
    chunk = jnp.full((16,), gp >> 4, jnp.int32)
    colbase = (gp * _G) & 127
    for lane in range(_G):
        lu = jnp.full((16,), uvec[lane] & 127, jnp.int32)
        li = jnp.full((16,), ivec[lane] & 127, jnp.int32)
        ucol = plsc.load_gather(upanels.at[lane], [iota16, lu])
        icol = plsc.load_gather(ipanels.at[lane], [iota16, li])
        col = jnp.full((16,), colbase + lane, jnp.int32)
        plsc.store_scatter(stage, [chunk, iota16, col], ucol * icol)


def _gmf_body(utt, itt, uids, iids, out_hbm,
              uidx_v, iidx_v, pu, pi, stage, sem0, sem1, sem2):
    wid = lax.axis_index("s") * _NC + lax.axis_index("c")
    base = wid * _BPW

    pltpu.sync_copy(uids.at[wid], uidx_v)
    pltpu.sync_copy(iids.at[wid], iidx_v)

    iota16 = lax.iota(jnp.int32, 16)
    sems = (sem0, sem1, sem2)

    def step(t, carry):
        for half in range(3):
            g = 3 * t + half

            @pl.when(g < _NG)
            def _():
                uvec = uidx_v[pl.ds(g * _G, 16)]
                ivec = iidx_v[pl.ds(g * _G, 16)]
                _fire(g, utt, uvec, pu.at[half], sems[half])
                _fire(g, itt, ivec, pi.at[half], sems[half])

            prev = (half + 2) % 3

            @pl.when((g > 0) & (g - 1 < _NG))
            def _():
                gp = g - 1
                _drain(utt, pu.at[prev], sems[prev])
                _drain(itt, pi.at[prev], sems[prev])
                upvec = uidx_v[pl.ds(gp * _G, 16)]
                ipvec = iidx_v[pl.ds(gp * _G, 16)]
                _process(gp, upvec, ipvec, pu.at[prev], pi.at[prev],
                         stage, iota16)
        return carry

    lax.fori_loop(0, (_NG + 2) // 3, step, 0)

    for ch in range(4):
        pltpu.sync_copy(
            stage.at[ch],
            out_hbm.at[:, pl.ds(pl.multiple_of(base + ch * 128, 128), 128)])


@jax.jit
def _gmf(utt, itt, uids, iids):
    run = functools.partial(
        pl.kernel,
        mesh=plsc.VectorSubcoreMesh(core_axis_name="c", subcore_axis_name="s"),
        out_type=jax.ShapeDtypeStruct((_D, _B), jnp.float32),
        scratch_types=[
            pltpu.VMEM((_IDP,), jnp.int32),
            pltpu.VMEM((_IDP,), jnp.int32),
            pltpu.VMEM((3, _G, _D, 128), jnp.float32),
            pltpu.VMEM((3, _G, _D, 128), jnp.float32),
            pltpu.VMEM((4, _D, 128), jnp.float32),
            pltpu.SemaphoreType.DMA,
            pltpu.SemaphoreType.DMA,
            pltpu.SemaphoreType.DMA,
        ],
        compiler_params=pltpu.CompilerParams(needs_layout_passes=False),
    )(_gmf_body)
    return run(utt, itt, uids, iids)


def kernel(user_ids, item_ids, user_table, item_table):
    uids = user_ids.astype(jnp.int32).reshape(_NW, _BPW)
    iids = item_ids.astype(jnp.int32).reshape(_NW, _BPW)
    uids = jnp.pad(uids, ((0, 0), (0, _IDP - _BPW)))
    iids = jnp.pad(iids, ((0, 0), (0, _IDP - _BPW)))
    out_t = _gmf(user_table.T, item_table.T, uids, iids)
    return out_t.T
